# baseline (speedup 1.0000x reference)
import jax
import jax.numpy as jnp
from jax.experimental import pallas as pl
from jax.experimental.pallas import tpu as pltpu

N = 10000
NFEAT = 128
NHID = 128
NCLASS = 64

BM = 400


def _xw1_body(x_ref, w1_ref, s1_ref):
    s1_ref[...] = jnp.dot(x_ref[...], w1_ref[...],
                          preferred_element_type=jnp.float32)


def _layer1_body(adj_ref, s1_ref, b1_ref, w2_ref, s2_ref):
    h = jnp.dot(adj_ref[...], s1_ref[...],
                preferred_element_type=jnp.float32)
    h = jnp.maximum(h + b1_ref[...], 0.0)
    s2_ref[...] = jnp.dot(h, w2_ref[...],
                          preferred_element_type=jnp.float32)


def _layer2_body(adj_ref, s2_ref, b2_ref, out_ref):
    h = jnp.dot(adj_ref[...], s2_ref[...],
                preferred_element_type=jnp.float32)
    h = h + b2_ref[...]
    m = jnp.max(h, axis=1, keepdims=True)
    e = jnp.exp(h - m)
    lse = jnp.log(jnp.sum(e, axis=1, keepdims=True))
    out_ref[...] = h - m - lse


def kernel(x, adj, W1, b1, W2, b2):
    nblk = N // BM
    b1r = b1.reshape(1, NHID)
    b2r = b2.reshape(1, NCLASS)

    s1 = pl.pallas_call(
        _xw1_body,
        out_shape=jax.ShapeDtypeStruct((N, NHID), jnp.float32),
    )(x, W1)

    s2 = pl.pallas_call(
        _layer1_body,
        grid=(nblk,),
        in_specs=[
            pl.BlockSpec((BM, N), lambda i: (i, 0)),
            pl.BlockSpec((N, NHID), lambda i: (0, 0)),
            pl.BlockSpec((1, NHID), lambda i: (0, 0)),
            pl.BlockSpec((NHID, NCLASS), lambda i: (0, 0)),
        ],
        out_specs=pl.BlockSpec((BM, NCLASS), lambda i: (i, 0)),
        out_shape=jax.ShapeDtypeStruct((N, NCLASS), jnp.float32),
        compiler_params=pltpu.CompilerParams(
            dimension_semantics=("parallel",),
        ),
    )(adj, s1, b1r, W2)

    out = pl.pallas_call(
        _layer2_body,
        grid=(nblk,),
        in_specs=[
            pl.BlockSpec((BM, N), lambda i: (i, 0)),
            pl.BlockSpec((N, NCLASS), lambda i: (0, 0)),
            pl.BlockSpec((1, NCLASS), lambda i: (0, 0)),
        ],
        out_specs=pl.BlockSpec((BM, NCLASS), lambda i: (i, 0)),
        out_shape=jax.ShapeDtypeStruct((N, NCLASS), jnp.float32),
        compiler_params=pltpu.CompilerParams(
            dimension_semantics=("parallel",),
        ),
    )(adj, s2, b2r)

    return out


# speedup vs baseline: 1.0530x; 1.0530x over previous
import jax
import jax.numpy as jnp
from jax.experimental import pallas as pl
from jax.experimental.pallas import tpu as pltpu

N = 10000
NFEAT = 128
NHID = 128
NCLASS = 64

BM = 400


def _gcn_body(x_ref, adj_ref, W1_ref, b1_ref, W2_ref, b2_ref,
              out_ref, s1_ref, s2_ref):
    p = pl.program_id(0)
    i = pl.program_id(1)

    @pl.when((p == 0) & (i == 0))
    def _prologue():
        s1_ref[...] = jnp.dot(x_ref[...], W1_ref[...],
                              preferred_element_type=jnp.float32)

    @pl.when(p == 0)
    def _sweep1():
        h = jnp.dot(adj_ref[...], s1_ref[...],
                    preferred_element_type=jnp.float32)
        h = jnp.maximum(h + b1_ref[...], 0.0)
        s2_ref[pl.ds(i * BM, BM), :] = jnp.dot(
            h, W2_ref[...], preferred_element_type=jnp.float32)

    @pl.when(p == 1)
    def _sweep2():
        h = jnp.dot(adj_ref[...], s2_ref[...],
                    preferred_element_type=jnp.float32)
        h = h + b2_ref[...]
        m = jnp.max(h, axis=1, keepdims=True)
        e = jnp.exp(h - m)
        lse = jnp.log(jnp.sum(e, axis=1, keepdims=True))
        out_ref[...] = h - m - lse


def kernel(x, adj, W1, b1, W2, b2):
    nblk = N // BM
    b1r = b1.reshape(1, NHID)
    b2r = b2.reshape(1, NCLASS)

    return pl.pallas_call(
        _gcn_body,
        grid=(2, nblk),
        in_specs=[
            pl.BlockSpec((N, NFEAT), lambda p, i: (0, 0)),
            pl.BlockSpec((BM, N), lambda p, i: (i, 0)),
            pl.BlockSpec((NFEAT, NHID), lambda p, i: (0, 0)),
            pl.BlockSpec((1, NHID), lambda p, i: (0, 0)),
            pl.BlockSpec((NHID, NCLASS), lambda p, i: (0, 0)),
            pl.BlockSpec((1, NCLASS), lambda p, i: (0, 0)),
        ],
        out_specs=pl.BlockSpec((BM, NCLASS), lambda p, i: (i * p, 0)),
        out_shape=jax.ShapeDtypeStruct((N, NCLASS), jnp.float32),
        scratch_shapes=[
            pltpu.VMEM((N, NHID), jnp.float32),
            pltpu.VMEM((N, NCLASS), jnp.float32),
        ],
        compiler_params=pltpu.CompilerParams(
            dimension_semantics=("arbitrary", "arbitrary"),
        ),
    )(x, adj, W1, b1r, W2, b2r)
